# Initial kernel scaffold; baseline (speedup 1.0000x reference)
import functools

import jax
import jax.numpy as jnp
from jax import lax
from jax.experimental import pallas as pl
from jax.experimental.pallas import tpu as pltpu
from jax.experimental.pallas import tpu_sc as plsc

N = 16 * 80 * 2048
NC, NS, L = 2, 16, 16
NW = NC * NS
PER_W = N // NW
CH = 16384
NCH = PER_W // CH
VPC = CH // L
UNROLL = 8

_mesh = plsc.VectorSubcoreMesh(core_axis_name="c", subcore_axis_name="s")


@functools.partial(
    pl.kernel,
    mesh=_mesh,
    out_type=jax.ShapeDtypeStruct((N,), jnp.int32),
    scratch_types=[
        pltpu.VMEM((2, CH), jnp.float32),
        pltpu.VMEM((2, CH), jnp.int32),
        pltpu.VMEM((L,), jnp.float32),
        pltpu.VMEM((L,), jnp.float32),
        pltpu.VMEM((L,), jnp.float32),
        pltpu.VMEM((L,), jnp.float32),
        pltpu.SemaphoreType.DMA,
        pltpu.SemaphoreType.DMA,
        pltpu.SemaphoreType.DMA,
        pltpu.SemaphoreType.DMA,
    ],
)
def _quantize_sc(x_hbm, mv_hbm, invd_hbm, dv_hbm, midb_hbm, out_hbm,
                 in_v, out_v, mv_v, invd_v, dv_v, midb_v,
                 sem_in0, sem_in1, sem_out0, sem_out1):
    wid = lax.axis_index("s") * NC + lax.axis_index("c")
    base = wid * PER_W

    pltpu.sync_copy(mv_hbm, mv_v)
    pltpu.sync_copy(invd_hbm, invd_v)
    pltpu.sync_copy(dv_hbm, dv_v)
    pltpu.sync_copy(midb_hbm, midb_v)

    mv = mv_v[:]
    invd = invd_v[:]
    dv = dv_v[:]
    midb = midb_v[:]

    sems_in = (sem_in0, sem_in1)
    sems_out = (sem_out0, sem_out1)

    def start_in(i):
        return pltpu.async_copy(
            x_hbm.at[pl.ds(base + i * CH, CH)], in_v.at[i % 2], sems_in[i % 2])

    def start_out(i):
        return pltpu.async_copy(
            out_v.at[i % 2], out_hbm.at[pl.ds(base + i * CH, CH)],
            sems_out[i % 2])

    def compute_chunk(buf):
        def body(j, carry):
            for u in range(UNROLL):
                off = (j * UNROLL + u) * L
                v = in_v[buf, pl.ds(off, L)]
                t = (v - mv) * invd
                t = jnp.minimum(jnp.maximum(t, 0.0), 14.0)
                k0 = t.astype(jnp.int32)
                mid = midb + k0.astype(jnp.float32) * dv
                out_v[buf, pl.ds(off, L)] = k0 + (v > mid).astype(jnp.int32)
            return carry
        lax.fori_loop(0, VPC // UNROLL, body, 0)

    pend_in = {0: start_in(0)}
    pend_out = {}
    for i in range(NCH):
        if i + 1 < NCH:
            pend_in[i + 1] = start_in(i + 1)
        pend_in.pop(i).wait()
        if i - 2 >= 0:
            pend_out.pop(i - 2).wait()
        compute_chunk(i % 2)
        pend_out[i] = start_out(i)
    pend_out.pop(NCH - 2).wait()
    pend_out.pop(NCH - 1).wait()


def kernel(x, codebook):
    m = codebook[0]
    d = codebook[1] - codebook[0]
    mv = jnp.full((L,), m, jnp.float32)
    invdv = jnp.full((L,), 1.0 / d, jnp.float32)
    dv = jnp.full((L,), d, jnp.float32)
    midbv = jnp.full((L,), m + 0.5 * d, jnp.float32)
    out = _quantize_sc(x.reshape(-1), mv, invdv, dv, midbv)
    return out.reshape(x.shape)


# speedup vs baseline: 3.1211x; 3.1211x over previous
import functools

import jax
import jax.numpy as jnp
from jax import lax
from jax.experimental import pallas as pl
from jax.experimental.pallas import tpu as pltpu
from jax.experimental.pallas import tpu_sc as plsc

N = 16 * 80 * 2048
NC, NS, L = 2, 16, 16
NW = NC * NS
PER_W = N // NW
CH = 16384
NCH = PER_W // CH
VPC = CH // L
UNROLL = 8

_mesh = plsc.VectorSubcoreMesh(core_axis_name="c", subcore_axis_name="s")


@functools.partial(
    pl.kernel,
    mesh=_mesh,
    out_type=jax.ShapeDtypeStruct((N,), jnp.int32),
    scratch_types=[
        pltpu.VMEM((2, CH), jnp.float32),
        pltpu.VMEM((2, CH), jnp.int32),
        pltpu.VMEM((L,), jnp.float32),
        pltpu.VMEM((L,), jnp.float32),
        pltpu.VMEM((L,), jnp.float32),
        pltpu.VMEM((L,), jnp.float32),
        pltpu.SemaphoreType.DMA,
        pltpu.SemaphoreType.DMA,
        pltpu.SemaphoreType.DMA,
        pltpu.SemaphoreType.DMA,
    ],
)
def _quantize_sc(x_hbm, mv_hbm, invd_hbm, dv_hbm, midb_hbm, out_hbm,
                 in_v, out_v, mv_v, invd_v, dv_v, midb_v,
                 sem_in0, sem_in1, sem_out0, sem_out1):
    wid = lax.axis_index("s") * NC + lax.axis_index("c")
    base = wid * PER_W

    pltpu.sync_copy(mv_hbm, mv_v)
    pltpu.sync_copy(invd_hbm, invd_v)
    pltpu.sync_copy(dv_hbm, dv_v)
    pltpu.sync_copy(midb_hbm, midb_v)

    mv = mv_v[:]
    invd = invd_v[:]
    dv = dv_v[:]
    midb = midb_v[:]

    sems_in = (sem_in0, sem_in1)
    sems_out = (sem_out0, sem_out1)

    def start_in(i):
        return pltpu.async_copy(
            x_hbm.at[pl.ds(base + i * CH, CH)], in_v.at[i % 2], sems_in[i % 2])

    def start_out(i):
        return pltpu.async_copy(
            out_v.at[i % 2], out_hbm.at[pl.ds(base + i * CH, CH)],
            sems_out[i % 2])

    def compute_chunk(buf):
        def body(j, carry):
            for u in range(UNROLL):
                off = (j * UNROLL + u) * L
                v = in_v[buf, pl.ds(off, L)]
                t = (v - mv) * invd
                t = jnp.minimum(jnp.maximum(t, 0.0), 14.0)
                k0 = t.astype(jnp.int32)
                mid = midb + k0.astype(jnp.float32) * dv
                out_v[buf, pl.ds(off, L)] = jnp.where(v > mid, k0 + 1, k0)
            return carry
        lax.fori_loop(0, VPC // UNROLL, body, 0)

    pend_in = {0: start_in(0)}
    pend_out = {}
    for i in range(NCH):
        if i + 1 < NCH:
            pend_in[i + 1] = start_in(i + 1)
        pend_in.pop(i).wait()
        if i - 2 >= 0:
            pend_out.pop(i - 2).wait()
        compute_chunk(i % 2)
        pend_out[i] = start_out(i)
    pend_out.pop(NCH - 2).wait()
    pend_out.pop(NCH - 1).wait()


def kernel(x, codebook):
    m = codebook[0]
    d = codebook[1] - codebook[0]
    mv = jnp.full((L,), m, jnp.float32)
    invdv = jnp.full((L,), 1.0 / d, jnp.float32)
    dv = jnp.full((L,), d, jnp.float32)
    midbv = jnp.full((L,), m + 0.5 * d, jnp.float32)
    out = _quantize_sc(x.reshape(-1), mv, invdv, dv, midbv)
    return out.reshape(x.shape)
